# Initial kernel scaffold; baseline (speedup 1.0000x reference)
import jax
import jax.numpy as jnp
from jax.experimental import pallas as pl


def kernel(t, z_nodes, edge_index, W1, b1, gamma1, beta1, W2, b2):
    raise NotImplementedError("write your pallas kernel here")


# speedup vs baseline: 9.3068x; 9.3068x over previous
import functools

import jax
import jax.numpy as jnp
from jax import lax
from jax.experimental import pallas as pl
from jax.experimental.pallas import tpu as pltpu
from jax.experimental.pallas import tpu_sc as plsc

N = 10000
D = 128
E = 320000

NC = 2
NS = 16
NW = NC * NS

K = 128
NCT = 80
CHUNKS = NW * NCT
E_PAD = CHUNKS * K
N_PAD = 10240
RPT = N_PAD // NS
DL = 16

_mesh = plsc.VectorSubcoreMesh(
    core_axis_name="c", subcore_axis_name="s", num_cores=NC, num_subcores=NS
)


def _wid(c, s):
    return s * NC + c


HALF = N // 2
HPAD = 5120
RPA = HPAD // NS
NCT2 = CHUNKS // NS


@functools.partial(
    pl.kernel,
    out_type=jax.ShapeDtypeStruct((NC, HPAD, D), jnp.float32),
    mesh=_mesh,
    scratch_types=[
        pltpu.VMEM((NCT2, K), jnp.int32),
        pltpu.VMEM((NCT2, K), jnp.int32),
        pltpu.VMEM((K,), jnp.int32),
        pltpu.VMEM((K, D), jnp.float32),
        pltpu.VMEM((K, D), jnp.float32),
        pltpu.VMEM_SHARED((HPAD, D), jnp.float32),
        pltpu.SemaphoreType.DMA,
        pltpu.SemaphoreType.DMA,
    ],
)
def _agg(g_hbm, src_hbm, dst_hbm, out_hbm, sidx, didx, didx1,
         rows0, rows1, acc, sem0, sem1):
    c = lax.axis_index("c")
    s = lax.axis_index("s")

    def fz(i, _):
        def fzl(l, _):
            rows1[i, pl.ds(l * 16, 16)] = jnp.zeros((16,), jnp.float32)
            return 0

        lax.fori_loop(0, D // 16, fzl, 0)
        return 0

    lax.fori_loop(0, 64, fz, 0)

    def initb(r, _):
        pltpu.sync_copy(rows1.at[pl.ds(0, 64), :],
                        acc.at[pl.ds(s * RPA + r * 64, 64), :])
        return 0

    lax.fori_loop(0, RPA // 64, initb, 0)
    pltpu.sync_copy(src_hbm.at[pl.ds(s * NCT2, NCT2), :], sidx)
    pltpu.sync_copy(dst_hbm.at[c, pl.ds(s * NCT2, NCT2), :], didx)
    plsc.subcore_barrier()

    def body(i, _):
        pltpu.async_copy(g_hbm.at[sidx.at[i]], rows0, sem0)

        def mv(l, _):
            didx1[pl.ds(l * 16, 16)] = didx[i, pl.ds(l * 16, 16)]
            return 0

        lax.fori_loop(0, K // 16, mv, 0)
        pltpu.make_async_copy(g_hbm.at[sidx.at[i]], rows0, sem0).wait()
        pltpu.sync_copy(rows0, acc.at[didx1], add=True)
        return 0

    lax.fori_loop(0, NCT2, body, 0)
    plsc.subcore_barrier()

    def obody(r, _):
        blk = rows0.at[pl.ds(0, 64), :]
        pltpu.sync_copy(acc.at[pl.ds(s * RPA + r * 64, 64), :], blk)
        pltpu.sync_copy(blk, out_hbm.at[c, pl.ds(s * RPA + r * 64, 64), :])
        return 0

    lax.fori_loop(0, RPA // 64, obody, 0)


def _dinv_of(degp_ref):
    return lax.rsqrt(_unsplit(degp_ref) + 1.0)


def _unsplit(p_ref):
    return jnp.concatenate([p_ref[0, :HALF, :], p_ref[1, :HALF, :]], axis=0)


def _tc1_body(z_ref, w1_ref, degp_ref, g1_ref):
    dinv = _dinv_of(degp_ref)
    h = jnp.dot(z_ref[...], w1_ref[...], preferred_element_type=jnp.float32)
    g1_ref[...] = h * dinv


def _tc2_body(p_ref, g1_ref, degp_ref, gamma_ref, beta_ref, b1_ref, w2_ref,
              g2_ref):
    dinv = _dinv_of(degp_ref)
    agg = _unsplit(p_ref) + g1_ref[...]
    h = agg * dinv + b1_ref[...]
    mean = jnp.mean(h, axis=0, keepdims=True)
    var = jnp.mean((h - mean) ** 2, axis=0, keepdims=True)
    hn = (h - mean) * lax.rsqrt(var + 1e-5) * gamma_ref[...] + beta_ref[...]
    hn = jnp.maximum(hn, 0.0)
    h2 = jnp.dot(hn, w2_ref[...], preferred_element_type=jnp.float32)
    g2_ref[...] = h2 * dinv


def _tc3_body(p_ref, g2_ref, degp_ref, b2_ref, out_ref):
    dinv = _dinv_of(degp_ref)
    agg = _unsplit(p_ref) + g2_ref[...]
    out_ref[...] = agg * dinv + b2_ref[...]


_g_shape = jax.ShapeDtypeStruct((N, D), jnp.float32)
_tc1 = pl.pallas_call(_tc1_body, out_shape=_g_shape)
_tc2 = pl.pallas_call(_tc2_body, out_shape=_g_shape)
_tc3 = pl.pallas_call(_tc3_body, out_shape=_g_shape)


def kernel(t, z_nodes, edge_index, W1, b1, gamma1, beta1, W2, b2):
    del t
    npad = E_PAD - E
    lane = lax.rem(jnp.arange(npad, dtype=jnp.int32), jnp.int32(16))
    src_p = jnp.concatenate([edge_index[0], lane]).reshape(CHUNKS, K)
    dstf = jnp.concatenate([edge_index[1], jnp.int32(N) + lane])
    sac = jnp.int32(HALF) + lax.rem(
        jnp.arange(E_PAD, dtype=jnp.int32), jnp.int32(16))
    d0 = jnp.where(dstf < HALF, dstf, sac)
    d1 = jnp.where((dstf >= HALF) & (dstf < N), dstf - HALF, sac)
    dst_loc = jnp.stack([d0, d1]).reshape(NC, CHUNKS, K)
    b1r = b1[None, :]
    b2r = b2[None, :]
    gammar = gamma1[None, :]
    betar = beta1[None, :]

    ones2d = jnp.ones((N, D), jnp.float32)
    degp = _agg(ones2d, src_p, dst_loc)
    g1 = _tc1(z_nodes, W1, degp)
    p1 = _agg(g1, src_p, dst_loc)
    g2 = _tc2(p1, g1, degp, gammar, betar, b1r, W2)
    p2 = _agg(g2, src_p, dst_loc)
    return _tc3(p2, g2, degp, b2r)


# speedup vs baseline: 16.6023x; 1.7839x over previous
import functools

import jax
import jax.numpy as jnp
from jax import lax
from jax.experimental import pallas as pl
from jax.experimental.pallas import tpu as pltpu
from jax.experimental.pallas import tpu_sc as plsc

N = 10000
D = 128
E = 320000

NC = 2
NS = 16
NW = NC * NS

K = 128
NCT = 80
CHUNKS = NW * NCT
E_PAD = CHUNKS * K
N_PAD = 10240
RPT = N_PAD // NS
DL = 16

_mesh = plsc.VectorSubcoreMesh(
    core_axis_name="c", subcore_axis_name="s", num_cores=NC, num_subcores=NS
)


def _wid(c, s):
    return s * NC + c


HALF = N // 2
HPAD = 5120
RPA = HPAD // NS
NCT2 = CHUNKS // NS


@functools.partial(
    pl.kernel,
    out_type=jax.ShapeDtypeStruct((NC, HPAD, D), jnp.float32),
    mesh=_mesh,
    scratch_types=[
        pltpu.VMEM((NCT2, K), jnp.int32),
        pltpu.VMEM((NCT2, K), jnp.int32),
        pltpu.VMEM((K,), jnp.int32),
        pltpu.VMEM((K,), jnp.int32),
        pltpu.VMEM((K, D), jnp.float32),
        pltpu.VMEM((K, D), jnp.float32),
        pltpu.VMEM_SHARED((HPAD, D), jnp.float32),
        pltpu.SemaphoreType.DMA,
        pltpu.SemaphoreType.DMA,
    ],
)
def _agg(g_hbm, src_hbm, dst_hbm, out_hbm, sidx, didx, didx1, didx2,
         rows0, rows1, acc, sem0, sem1):
    c = lax.axis_index("c")
    s = lax.axis_index("s")

    def fz(i, _):
        def fzl(l, _):
            rows1[i, pl.ds(l * 16, 16)] = jnp.zeros((16,), jnp.float32)
            return 0

        lax.fori_loop(0, D // 16, fzl, 0)
        return 0

    lax.fori_loop(0, 64, fz, 0)

    def initb(r, _):
        pltpu.sync_copy(rows1.at[pl.ds(0, 64), :],
                        acc.at[pl.ds(s * RPA + r * 64, 64), :])
        return 0

    lax.fori_loop(0, RPA // 64, initb, 0)
    pltpu.sync_copy(src_hbm.at[pl.ds(s * NCT2, NCT2), :], sidx)
    pltpu.sync_copy(dst_hbm.at[c, pl.ds(s * NCT2, NCT2), :], didx)
    plsc.subcore_barrier()

    pltpu.async_copy(g_hbm.at[sidx.at[0]], rows0, sem0)

    def _mv(dst1, i):
        def mv(l, _):
            dst1[pl.ds(l * 16, 16)] = didx[i, pl.ds(l * 16, 16)]
            return 0

        lax.fori_loop(0, K // 16, mv, 0)

    def _drain(i, rows, sem, dst1):
        pltpu.make_async_copy(g_hbm.at[sidx.at[i]], rows, sem).wait()
        pltpu.sync_copy(rows, acc.at[dst1], add=True)

    def body(j, _):
        i = 2 * j
        pltpu.async_copy(g_hbm.at[sidx.at[i + 1]], rows1, sem1)
        _mv(didx1, i)
        _drain(i, rows0, sem0, didx1)
        pltpu.async_copy(g_hbm.at[sidx.at[i + 2]], rows0, sem0)
        _mv(didx2, i + 1)
        _drain(i + 1, rows1, sem1, didx2)
        return 0

    lax.fori_loop(0, NCT2 // 2 - 1, body, 0)
    it = NCT2 - 2
    pltpu.async_copy(g_hbm.at[sidx.at[it + 1]], rows1, sem1)
    _mv(didx1, it)
    _drain(it, rows0, sem0, didx1)
    _mv(didx2, it + 1)
    _drain(it + 1, rows1, sem1, didx2)
    plsc.subcore_barrier()

    def obody(r, _):
        blk = rows0.at[pl.ds(0, 64), :]
        pltpu.sync_copy(acc.at[pl.ds(s * RPA + r * 64, 64), :], blk)
        pltpu.sync_copy(blk, out_hbm.at[c, pl.ds(s * RPA + r * 64, 64), :])
        return 0

    lax.fori_loop(0, RPA // 64, obody, 0)


@functools.partial(
    pl.kernel,
    out_type=jax.ShapeDtypeStruct((NC, HPAD, D), jnp.float32),
    mesh=_mesh,
    scratch_types=[
        pltpu.VMEM((NCT2, K), jnp.int32),
        pltpu.VMEM((K,), jnp.int32),
        pltpu.VMEM((K, D), jnp.float32),
        pltpu.VMEM((64, D), jnp.float32),
        pltpu.VMEM_SHARED((HPAD, D), jnp.float32),
    ],
)
def _degk(dst_hbm, out_hbm, didx, didx1, ones, zb, acc):
    c = lax.axis_index("c")
    s = lax.axis_index("s")

    def fo(i, _):
        def fol(l, _):
            ones[i, pl.ds(l * 16, 16)] = jnp.full((16,), 1.0, jnp.float32)
            return 0

        lax.fori_loop(0, D // 16, fol, 0)
        return 0

    lax.fori_loop(0, K, fo, 0)

    def fz(i, _):
        def fzl(l, _):
            zb[i, pl.ds(l * 16, 16)] = jnp.zeros((16,), jnp.float32)
            return 0

        lax.fori_loop(0, D // 16, fzl, 0)
        return 0

    lax.fori_loop(0, 64, fz, 0)

    def initb(r, _):
        pltpu.sync_copy(zb, acc.at[pl.ds(s * RPA + r * 64, 64), :])
        return 0

    lax.fori_loop(0, RPA // 64, initb, 0)
    pltpu.sync_copy(dst_hbm.at[c, pl.ds(s * NCT2, NCT2), :], didx)
    plsc.subcore_barrier()

    def body(i, _):
        def mv(l, _):
            didx1[pl.ds(l * 16, 16)] = didx[i, pl.ds(l * 16, 16)]
            return 0

        lax.fori_loop(0, K // 16, mv, 0)
        pltpu.sync_copy(ones, acc.at[didx1], add=True)
        return 0

    lax.fori_loop(0, NCT2, body, 0)
    plsc.subcore_barrier()

    def obody(r, _):
        pltpu.sync_copy(acc.at[pl.ds(s * RPA + r * 64, 64), :], zb)
        pltpu.sync_copy(zb, out_hbm.at[c, pl.ds(s * RPA + r * 64, 64), :])
        return 0

    lax.fori_loop(0, RPA // 64, obody, 0)


def _dinv_of(degp_ref):
    return lax.rsqrt(_unsplit(degp_ref) + 1.0)


def _unsplit(p_ref):
    return jnp.concatenate([p_ref[0, :HALF, :], p_ref[1, :HALF, :]], axis=0)


def _tc1_body(z_ref, w1_ref, degp_ref, g1_ref):
    dinv = _dinv_of(degp_ref)
    h = jnp.dot(z_ref[...], w1_ref[...], preferred_element_type=jnp.float32)
    g1_ref[...] = h * dinv


def _tc2_body(p_ref, g1_ref, degp_ref, gamma_ref, beta_ref, b1_ref, w2_ref,
              g2_ref):
    dinv = _dinv_of(degp_ref)
    agg = _unsplit(p_ref) + g1_ref[...]
    h = agg * dinv + b1_ref[...]
    mean = jnp.mean(h, axis=0, keepdims=True)
    var = jnp.mean((h - mean) ** 2, axis=0, keepdims=True)
    hn = (h - mean) * lax.rsqrt(var + 1e-5) * gamma_ref[...] + beta_ref[...]
    hn = jnp.maximum(hn, 0.0)
    h2 = jnp.dot(hn, w2_ref[...], preferred_element_type=jnp.float32)
    g2_ref[...] = h2 * dinv


def _tc3_body(p_ref, g2_ref, degp_ref, b2_ref, out_ref):
    dinv = _dinv_of(degp_ref)
    agg = _unsplit(p_ref) + g2_ref[...]
    out_ref[...] = agg * dinv + b2_ref[...]


_g_shape = jax.ShapeDtypeStruct((N, D), jnp.float32)
_tc1 = pl.pallas_call(_tc1_body, out_shape=_g_shape)
_tc2 = pl.pallas_call(_tc2_body, out_shape=_g_shape)
_tc3 = pl.pallas_call(_tc3_body, out_shape=_g_shape)


def kernel(t, z_nodes, edge_index, W1, b1, gamma1, beta1, W2, b2):
    del t
    npad = E_PAD - E
    lane = lax.rem(jnp.arange(npad, dtype=jnp.int32), jnp.int32(16))
    src_p = jnp.concatenate([edge_index[0], lane]).reshape(CHUNKS, K)
    dstf = jnp.concatenate([edge_index[1], jnp.int32(N) + lane])
    sac = jnp.int32(HALF) + lax.rem(
        jnp.arange(E_PAD, dtype=jnp.int32), jnp.int32(16))
    d0 = jnp.where(dstf < HALF, dstf, sac)
    d1 = jnp.where((dstf >= HALF) & (dstf < N), dstf - HALF, sac)
    dst_loc = jnp.stack([d0, d1]).reshape(NC, CHUNKS, K)
    b1r = b1[None, :]
    b2r = b2[None, :]
    gammar = gamma1[None, :]
    betar = beta1[None, :]

    degp = _degk(dst_loc)
    g1 = _tc1(z_nodes, W1, degp)
    p1 = _agg(g1, src_p, dst_loc)
    g2 = _tc2(p1, g1, degp, gammar, betar, b1r, W2)
    p2 = _agg(g2, src_p, dst_loc)
    return _tc3(p2, g2, degp, b2r)
